# baseline (speedup 1.0000x reference)
import functools

import jax
import jax.numpy as jnp
from jax.experimental import pallas as pl
from jax.experimental.pallas import tpu as pltpu


def _blk(n, target):
    if n <= target:
        return n
    for b in range(target, 0, -1):
        if n % b == 0 and b % 8 == 0:
            return b
    return n


def _proj_kernel(feat_ref, shuf_ref, w1_ref, w3_ref, sa_ref, sd_ref):
    f = feat_ref[...]
    s = shuf_ref[...]
    w1 = w1_ref[...]
    w3 = w3_ref[...]
    sa_ref[...] = jnp.concatenate(
        [jnp.dot(f, w1, preferred_element_type=jnp.float32),
         jnp.dot(s, w1, preferred_element_type=jnp.float32)], axis=1)
    sd_ref[...] = jnp.concatenate(
        [jnp.dot(f, w3, preferred_element_type=jnp.float32),
         jnp.dot(s, w3, preferred_element_type=jnp.float32)], axis=1)


def _pass1_kernel(n, bm, bk, adj_ref, diff_ref, sa_ref, sd_ref, msk_ref,
                  b1t_ref, b3t_ref, w2_ref, w4_ref, pa_ref,
                  ha_ref, hd_ref, u_ref, racc_ref, acc_a, acc_d):
    i = pl.program_id(0)
    k = pl.program_id(1)
    nk = pl.num_programs(1)

    @pl.when(k == 0)
    def _():
        acc_a[...] = jnp.zeros_like(acc_a)
        acc_d[...] = jnp.zeros_like(acc_d)

    sa_b = sa_ref[pl.ds(k * bk, bk), :]
    sd_b = sd_ref[pl.ds(k * bk, bk), :]

    def accum(adj_b, diff_b):
        acc_a[...] += jnp.dot(adj_b, sa_b, preferred_element_type=jnp.float32)
        acc_d[...] += jnp.dot(diff_b, sd_b, preferred_element_type=jnp.float32)

    ragged = (n % bk != 0)

    @pl.when(k < nk - 1 if ragged else k >= 0)
    def _():
        accum(adj_ref[...], diff_ref[...])

    if ragged:
        @pl.when(k == nk - 1)
        def _():
            rem = n - (nk - 1) * bk
            lane = jax.lax.broadcasted_iota(jnp.int32, (bm, bk), 1)
            accum(jnp.where(lane < rem, adj_ref[...], 0.0),
                  jnp.where(lane < rem, diff_ref[...], 0.0))

    @pl.when(k == nk - 1)
    def _():
        a = pa_ref[0, 0]
        xa = acc_a[...] + b1t_ref[...]
        xd = acc_d[...] + b3t_ref[...]
        ha = jnp.where(xa > 0, xa, a * xa)
        hd = jnp.where(xd > 0, xd, a * xd)
        ha_ref[...] = ha
        hd_ref[...] = hd
        u_ref[...] = jnp.concatenate(
            [jnp.dot(ha[:, :128], w2_ref[...],
                     preferred_element_type=jnp.float32),
             jnp.dot(hd[:, :128], w4_ref[...],
                     preferred_element_type=jnp.float32)], axis=1)
        m = msk_ref[0]
        pa = jnp.dot(m, ha, preferred_element_type=jnp.float32)
        pd = jnp.dot(m, hd, preferred_element_type=jnp.float32)
        racc_ref[...] = jnp.concatenate(
            [pa[:, :128], pd[:, :128]], axis=1)[None]


def _pass2_kernel(n, bm, bk, ncls, adj_ref, diff_ref, u_ref, ha_ref, hd_ref,
                  racc_ref, wb_ref, b24_ref, bb_ref, inv_ref,
                  out_ref, sc_ref, acc):
    k = pl.program_id(1)
    nk = pl.num_programs(1)

    @pl.when(k == 0)
    def _():
        acc[...] = jnp.zeros_like(acc)

    u_b = u_ref[pl.ds(k * bk, bk), :]

    def accum(adj_b, diff_b):
        acc[...] += (jnp.dot(adj_b, u_b[:, :ncls],
                             preferred_element_type=jnp.float32)
                     + jnp.dot(diff_b, u_b[:, ncls:],
                               preferred_element_type=jnp.float32))

    ragged = (n % bk != 0)

    @pl.when(k < nk - 1 if ragged else k >= 0)
    def _():
        accum(adj_ref[...], diff_ref[...])

    if ragged:
        @pl.when(k == nk - 1)
        def _():
            rem = n - (nk - 1) * bk
            lane = jax.lax.broadcasted_iota(jnp.int32, (bm, bk), 1)
            accum(jnp.where(lane < rem, adj_ref[...], 0.0),
                  jnp.where(lane < rem, diff_ref[...], 0.0))

    @pl.when(k == nk - 1)
    def _():
        y = acc[...] + b24_ref[...]
        mx = jnp.max(y, axis=1, keepdims=True)
        z = y - mx
        out_ref[...] = z - jnp.log(jnp.sum(jnp.exp(z), axis=1, keepdims=True))

        rs = jnp.sum(racc_ref[...], axis=0)
        c = jax.nn.sigmoid(rs * inv_ref[0, 0])
        c1 = c[:, :128]
        c2 = c[:, 128:]
        wb = wb_ref[...]
        dn = (((1,), (1,)), ((), ()))
        v1 = jax.lax.dot_general(wb, c1, dn,
                                 preferred_element_type=jnp.float32)
        v2 = jax.lax.dot_general(wb, c2, dn,
                                 preferred_element_type=jnp.float32)
        h1 = ha_ref[:, :128]
        h3 = ha_ref[:, 128:]
        h2 = hd_ref[:, :128]
        h4 = hd_ref[:, 128:]
        t1 = jnp.dot(h2, v1, preferred_element_type=jnp.float32)
        t2 = jnp.dot(h1, v2, preferred_element_type=jnp.float32)
        t3 = jnp.dot(h4, v1, preferred_element_type=jnp.float32)
        t4 = jnp.dot(h3, v2, preferred_element_type=jnp.float32)
        sc_ref[...] = jnp.concatenate([t1, t2, t3, t4], axis=1) + bb_ref[0, 0]


def kernel(feature, adj, diff, shuf_fts, sparse, msk, samp_bias1, samp_bias2,
           W1, b1, W2, b2, W3, b3, W4, b4, Wb, bb, prelu_a):
    del sparse, samp_bias1, samp_bias2
    n, nfeat = feature.shape
    nhid = W1.shape[1]
    ncls = W2.shape[1]
    f32 = jnp.float32

    bm = _blk(n, 500)
    bk = min(1024, ((n + 127) // 128) * 128)
    ni = n // bm
    nkk = -(-n // bk)
    npad = nkk * bk
    bmc = _blk(n, 1000)
    bkc = min(2048, ((n + 127) // 128) * 128)
    nic = n // bmc
    nkc = -(-n // bkc)
    npadc = nkc * bkc

    b1t = jnp.concatenate([b1, b1]).reshape(1, 2 * nhid)
    b3t = jnp.concatenate([b3, b3]).reshape(1, 2 * nhid)
    b24 = (b2 + b4).reshape(1, ncls)
    wb0 = Wb[0]
    bb2 = bb.reshape(1, 1)
    pa2 = prelu_a.reshape(1, 1)
    msk3 = msk.reshape(ni, 1, bm)
    inv = (1.0 / (n * jnp.sum(msk))).reshape(1, 1).astype(f32)

    sa, sd = pl.pallas_call(
        _proj_kernel,
        grid=(ni,),
        in_specs=[
            pl.BlockSpec((bm, nfeat), lambda i: (i, 0)),
            pl.BlockSpec((bm, nfeat), lambda i: (i, 0)),
            pl.BlockSpec((nfeat, nhid), lambda i: (0, 0)),
            pl.BlockSpec((nfeat, nhid), lambda i: (0, 0)),
        ],
        out_specs=[
            pl.BlockSpec((bm, 2 * nhid), lambda i: (i, 0)),
            pl.BlockSpec((bm, 2 * nhid), lambda i: (i, 0)),
        ],
        out_shape=[
            jax.ShapeDtypeStruct((n, 2 * nhid), f32),
            jax.ShapeDtypeStruct((n, 2 * nhid), f32),
        ],
        compiler_params=pltpu.CompilerParams(
            dimension_semantics=("parallel",)),
    )(feature, shuf_fts, W1, W3)

    sa_p = jnp.pad(sa, ((0, npad - n), (0, 0)))
    sd_p = jnp.pad(sd, ((0, npad - n), (0, 0)))
    ha, hd, u, racc = pl.pallas_call(
        functools.partial(_pass1_kernel, n, bm, bk),
        grid=(ni, nkk),
        in_specs=[
            pl.BlockSpec((bm, bk), lambda i, k: (i, k)),
            pl.BlockSpec((bm, bk), lambda i, k: (i, k)),
            pl.BlockSpec((npad, 2 * nhid), lambda i, k: (0, 0)),
            pl.BlockSpec((npad, 2 * nhid), lambda i, k: (0, 0)),
            pl.BlockSpec((1, 1, bm), lambda i, k: (i, 0, 0)),
            pl.BlockSpec((1, 2 * nhid), lambda i, k: (0, 0)),
            pl.BlockSpec((1, 2 * nhid), lambda i, k: (0, 0)),
            pl.BlockSpec((nhid, ncls), lambda i, k: (0, 0)),
            pl.BlockSpec((nhid, ncls), lambda i, k: (0, 0)),
            pl.BlockSpec((1, 1), lambda i, k: (0, 0)),
        ],
        out_specs=[
            pl.BlockSpec((bm, 2 * nhid), lambda i, k: (i, 0)),
            pl.BlockSpec((bm, 2 * nhid), lambda i, k: (i, 0)),
            pl.BlockSpec((bm, 2 * ncls), lambda i, k: (i, 0)),
            pl.BlockSpec((1, 1, 2 * nhid), lambda i, k: (i, 0, 0)),
        ],
        out_shape=[
            jax.ShapeDtypeStruct((n, 2 * nhid), f32),
            jax.ShapeDtypeStruct((n, 2 * nhid), f32),
            jax.ShapeDtypeStruct((n, 2 * ncls), f32),
            jax.ShapeDtypeStruct((ni, 1, 2 * nhid), f32),
        ],
        scratch_shapes=[
            pltpu.VMEM((bm, 2 * nhid), f32),
            pltpu.VMEM((bm, 2 * nhid), f32),
        ],
        compiler_params=pltpu.CompilerParams(
            dimension_semantics=("parallel", "arbitrary")),
    )(adj, diff, sa_p, sd_p, msk3, b1t, b3t, W2, W4, pa2)

    u_p = jnp.pad(u, ((0, npadc - n), (0, 0)))
    out, sc = pl.pallas_call(
        functools.partial(_pass2_kernel, n, bmc, bkc, ncls),
        grid=(nic, nkc),
        in_specs=[
            pl.BlockSpec((bmc, bkc), lambda i, k: (i, k)),
            pl.BlockSpec((bmc, bkc), lambda i, k: (i, k)),
            pl.BlockSpec((npadc, 2 * ncls), lambda i, k: (0, 0)),
            pl.BlockSpec((bmc, 2 * nhid), lambda i, k: (i, 0)),
            pl.BlockSpec((bmc, 2 * nhid), lambda i, k: (i, 0)),
            pl.BlockSpec((ni, 1, 2 * nhid), lambda i, k: (0, 0, 0)),
            pl.BlockSpec((nhid, nhid), lambda i, k: (0, 0)),
            pl.BlockSpec((1, ncls), lambda i, k: (0, 0)),
            pl.BlockSpec((1, 1), lambda i, k: (0, 0)),
            pl.BlockSpec((1, 1), lambda i, k: (0, 0)),
        ],
        out_specs=[
            pl.BlockSpec((bmc, ncls), lambda i, k: (i, 0)),
            pl.BlockSpec((bmc, 4), lambda i, k: (i, 0)),
        ],
        out_shape=[
            jax.ShapeDtypeStruct((n, ncls), f32),
            jax.ShapeDtypeStruct((n, 4), f32),
        ],
        scratch_shapes=[
            pltpu.VMEM((bmc, ncls), f32),
        ],
        compiler_params=pltpu.CompilerParams(
            dimension_semantics=("parallel", "arbitrary")),
    )(adj, diff, u_p, ha, hd, racc, wb0, b24, bb2, inv)

    logits = sc.T.reshape(1, 4 * n)
    return (out, logits)


# speedup vs baseline: 1.2130x; 1.2130x over previous
import functools

import jax
import jax.numpy as jnp
from jax.experimental import pallas as pl
from jax.experimental.pallas import tpu as pltpu

_F32 = jnp.float32


def _blk(n, target):
    if n <= target:
        return n
    for b in range(target, 0, -1):
        if n % b == 0 and b % 8 == 0:
            return b
    return n


def _proj_kernel(feat_ref, shuf_ref, w1_ref, w3_ref, sa_ref, sd_ref):
    f = feat_ref[...]
    s = shuf_ref[...]
    w1 = w1_ref[...]
    w3 = w3_ref[...]
    sa_ref[...] = jnp.concatenate(
        [jnp.dot(f, w1, preferred_element_type=_F32),
         jnp.dot(s, w1, preferred_element_type=_F32)], axis=1)
    sd_ref[...] = jnp.concatenate(
        [jnp.dot(f, w3, preferred_element_type=_F32),
         jnp.dot(s, w3, preferred_element_type=_F32)], axis=1)


def _gconv1_kernel(nhid, a_ref, s_ref, msk_ref, bt_ref, wp_ref, pa_ref,
                   h_ref, u_ref, racc_ref):
    x = jnp.dot(a_ref[...], s_ref[...], preferred_element_type=_F32)
    x = x + bt_ref[...]
    a = pa_ref[0, 0]
    h = jnp.where(x > 0, x, a * x)
    h_ref[...] = h
    u_ref[...] = jnp.dot(h[:, :nhid], wp_ref[...], preferred_element_type=_F32)
    m = msk_ref[0]
    racc_ref[...] = jnp.dot(m, h, preferred_element_type=_F32)[:, :nhid][None]


def _gconv2a_kernel(a_ref, u_ref, p_ref):
    p_ref[...] = jnp.dot(a_ref[...], u_ref[...], preferred_element_type=_F32)


def _gconv2b_kernel(nhid, d_ref, u_ref, p_ref, ha_ref, hd_ref, ra_ref,
                    rd_ref, wb_ref, b24_ref, bb_ref, inv_ref,
                    out_ref, sc_ref):
    y = (p_ref[...]
         + jnp.dot(d_ref[...], u_ref[...], preferred_element_type=_F32)
         + b24_ref[...])
    mx = jnp.max(y, axis=1, keepdims=True)
    z = y - mx
    out_ref[...] = z - jnp.log(jnp.sum(jnp.exp(z), axis=1, keepdims=True))

    c1 = jax.nn.sigmoid(jnp.sum(ra_ref[...], axis=0) * inv_ref[0, 0])
    c2 = jax.nn.sigmoid(jnp.sum(rd_ref[...], axis=0) * inv_ref[0, 0])
    wb = wb_ref[...]
    dn = (((1,), (1,)), ((), ()))
    v1 = jax.lax.dot_general(wb, c1, dn, preferred_element_type=_F32)
    v2 = jax.lax.dot_general(wb, c2, dn, preferred_element_type=_F32)
    h1 = ha_ref[:, :nhid]
    h3 = ha_ref[:, nhid:]
    h2 = hd_ref[:, :nhid]
    h4 = hd_ref[:, nhid:]
    t1 = jnp.dot(h2, v1, preferred_element_type=_F32)
    t2 = jnp.dot(h1, v2, preferred_element_type=_F32)
    t3 = jnp.dot(h4, v1, preferred_element_type=_F32)
    t4 = jnp.dot(h3, v2, preferred_element_type=_F32)
    sc_ref[...] = jnp.concatenate([t1, t2, t3, t4], axis=1) + bb_ref[0, 0]


def kernel(feature, adj, diff, shuf_fts, sparse, msk, samp_bias1, samp_bias2,
           W1, b1, W2, b2, W3, b3, W4, b4, Wb, bb, prelu_a):
    del sparse, samp_bias1, samp_bias2
    n, nfeat = feature.shape
    nhid = W1.shape[1]
    ncls = W2.shape[1]

    bm = _blk(n, 400)
    ni = n // bm

    b1t = jnp.concatenate([b1, b1]).reshape(1, 2 * nhid)
    b3t = jnp.concatenate([b3, b3]).reshape(1, 2 * nhid)
    b24 = (b2 + b4).reshape(1, ncls)
    wb0 = Wb[0]
    bb2 = bb.reshape(1, 1)
    pa2 = prelu_a.reshape(1, 1)
    msk3 = msk.reshape(ni, 1, bm)
    inv = (1.0 / (n * jnp.sum(msk))).reshape(1, 1).astype(_F32)

    par = pltpu.CompilerParams(dimension_semantics=("parallel",))

    sa, sd = pl.pallas_call(
        _proj_kernel,
        grid=(ni,),
        in_specs=[
            pl.BlockSpec((bm, nfeat), lambda i: (i, 0)),
            pl.BlockSpec((bm, nfeat), lambda i: (i, 0)),
            pl.BlockSpec((nfeat, nhid), lambda i: (0, 0)),
            pl.BlockSpec((nfeat, nhid), lambda i: (0, 0)),
        ],
        out_specs=[
            pl.BlockSpec((bm, 2 * nhid), lambda i: (i, 0)),
            pl.BlockSpec((bm, 2 * nhid), lambda i: (i, 0)),
        ],
        out_shape=[
            jax.ShapeDtypeStruct((n, 2 * nhid), _F32),
            jax.ShapeDtypeStruct((n, 2 * nhid), _F32),
        ],
        compiler_params=par,
    )(feature, shuf_fts, W1, W3)

    def gconv1(mat, s, bt, wp):
        return pl.pallas_call(
            functools.partial(_gconv1_kernel, nhid),
            grid=(ni,),
            in_specs=[
                pl.BlockSpec((bm, n), lambda i: (i, 0)),
                pl.BlockSpec((n, 2 * nhid), lambda i: (0, 0)),
                pl.BlockSpec((1, 1, bm), lambda i: (i, 0, 0)),
                pl.BlockSpec((1, 2 * nhid), lambda i: (0, 0)),
                pl.BlockSpec((nhid, ncls), lambda i: (0, 0)),
                pl.BlockSpec((1, 1), lambda i: (0, 0)),
            ],
            out_specs=[
                pl.BlockSpec((bm, 2 * nhid), lambda i: (i, 0)),
                pl.BlockSpec((bm, ncls), lambda i: (i, 0)),
                pl.BlockSpec((1, 1, nhid), lambda i: (i, 0, 0)),
            ],
            out_shape=[
                jax.ShapeDtypeStruct((n, 2 * nhid), _F32),
                jax.ShapeDtypeStruct((n, ncls), _F32),
                jax.ShapeDtypeStruct((ni, 1, nhid), _F32),
            ],
            compiler_params=par,
        )(mat, s, msk3, bt, wp, pa2)

    ha, u1, ra = gconv1(adj, sa, b1t, W2)
    hd, u2, rd = gconv1(diff, sd, b3t, W4)

    partial = pl.pallas_call(
        _gconv2a_kernel,
        grid=(ni,),
        in_specs=[
            pl.BlockSpec((bm, n), lambda i: (i, 0)),
            pl.BlockSpec((n, ncls), lambda i: (0, 0)),
        ],
        out_specs=pl.BlockSpec((bm, ncls), lambda i: (i, 0)),
        out_shape=jax.ShapeDtypeStruct((n, ncls), _F32),
        compiler_params=par,
    )(adj, u1)

    out, sc = pl.pallas_call(
        functools.partial(_gconv2b_kernel, nhid),
        grid=(ni,),
        in_specs=[
            pl.BlockSpec((bm, n), lambda i: (i, 0)),
            pl.BlockSpec((n, ncls), lambda i: (0, 0)),
            pl.BlockSpec((bm, ncls), lambda i: (i, 0)),
            pl.BlockSpec((bm, 2 * nhid), lambda i: (i, 0)),
            pl.BlockSpec((bm, 2 * nhid), lambda i: (i, 0)),
            pl.BlockSpec((ni, 1, nhid), lambda i: (0, 0, 0)),
            pl.BlockSpec((ni, 1, nhid), lambda i: (0, 0, 0)),
            pl.BlockSpec((nhid, nhid), lambda i: (0, 0)),
            pl.BlockSpec((1, ncls), lambda i: (0, 0)),
            pl.BlockSpec((1, 1), lambda i: (0, 0)),
            pl.BlockSpec((1, 1), lambda i: (0, 0)),
        ],
        out_specs=[
            pl.BlockSpec((bm, ncls), lambda i: (i, 0)),
            pl.BlockSpec((bm, 4), lambda i: (i, 0)),
        ],
        out_shape=[
            jax.ShapeDtypeStruct((n, ncls), _F32),
            jax.ShapeDtypeStruct((n, 4), _F32),
        ],
        compiler_params=par,
    )(diff, u2, partial, ha, hd, ra, rd, wb0, b24, bb2, inv)

    logits = sc.T.reshape(1, 4 * n)
    return (out, logits)
